# Initial kernel scaffold; baseline (speedup 1.0000x reference)
import jax
import jax.numpy as jnp
from jax.experimental import pallas as pl


def kernel(h, edge_index, edge_weights, W0, b0, W1, b1, W2, b2, eps):
    raise NotImplementedError("write your pallas kernel here")


# speedup vs baseline: 2.9226x; 2.9226x over previous
import jax
import jax.numpy as jnp
from jax import lax
from jax.experimental import pallas as pl
from jax.experimental.pallas import tpu as pltpu
from jax.experimental.pallas import tpu_sc as plsc

NC = 2
NS = 16
L = 16
NW = NC * NS
B = 128
ZR = 128


def _zero_rows(rows, d):
  def zfill(r, carry):
    for j in range(d // L):
      rows[r, pl.ds(j * L, L)] = jnp.zeros((L,), jnp.float32)
    return carry
  lax.fori_loop(0, ZR, zfill, 0)


def _make_msgpass(n, d, epad):
  e_per_w = epad // NW
  nb = e_per_w // B
  rpt = n // NS
  nz = rpt // ZR
  mesh = plsc.VectorSubcoreMesh(core_axis_name="c", subcore_axis_name="s")

  def body(h_hbm, src_h, dst_h, ew_h, pooled_o,
           src_v, dst_v, ew_v, rows, pooled_sh, sem):
    c = lax.axis_index("c")
    s = lax.axis_index("s")
    wid = c * NS + s

    _zero_rows(rows, d)
    for z in range(nz):
      pltpu.sync_copy(rows, pooled_sh.at[pl.ds(s * rpt + z * ZR, ZR)])
    plsc.subcore_barrier()

    ebase = wid * e_per_w

    def batch(i, carry):
      base = ebase + i * B
      pltpu.sync_copy(src_h.at[pl.ds(base, B)], src_v)
      pltpu.sync_copy(dst_h.at[pl.ds(base, B)], dst_v)
      pltpu.sync_copy(ew_h.at[pl.ds(base, B)], ew_v)
      pltpu.async_copy(h_hbm.at[src_v], rows, sem).wait()

      def scale(k, kc):
        idx = jnp.full((L,), k, jnp.int32)
        w = plsc.load_gather(ew_v, [idx])
        for j in range(d // L):
          rows[k, pl.ds(j * L, L)] = rows[k, pl.ds(j * L, L)] * w
        return kc

      lax.fori_loop(0, B, scale, 0)
      pltpu.sync_copy(rows, pooled_sh.at[dst_v], add=True)
      return carry

    lax.fori_loop(0, nb, batch, 0)
    plsc.subcore_barrier()

    row0 = c * n + s * rpt
    for z in range(nz):
      pltpu.sync_copy(pooled_sh.at[pl.ds(s * rpt + z * ZR, ZR)], rows)
      pltpu.sync_copy(rows, pooled_o.at[pl.ds(row0 + z * ZR, ZR)])

  return pl.kernel(
      body,
      out_type=jax.ShapeDtypeStruct((NC * n, d), jnp.float32),
      mesh=mesh,
      scratch_types=[
          pltpu.VMEM((B,), jnp.int32),
          pltpu.VMEM((B,), jnp.int32),
          pltpu.VMEM((B,), jnp.float32),
          pltpu.VMEM((B, d), jnp.float32),
          pltpu.VMEM_SHARED((n, d), jnp.float32),
          pltpu.SemaphoreType.DMA,
      ],
      compiler_params=pltpu.CompilerParams(needs_layout_passes=False))


def _make_degree(n, d, epad):
  e_per_w = epad // NW
  nb = e_per_w // B
  rpt = n // NS
  nz = rpt // ZR
  mesh = plsc.VectorSubcoreMesh(core_axis_name="c", subcore_axis_name="s")

  def body(dst_h, cnt_h, deg_o, dst_v, cnt_v, rows, deg_sh):
    c = lax.axis_index("c")
    s = lax.axis_index("s")
    wid = c * NS + s

    _zero_rows(rows, d)
    for z in range(nz):
      pltpu.sync_copy(rows, deg_sh.at[pl.ds(s * rpt + z * ZR, ZR)])
    plsc.subcore_barrier()

    ebase = wid * e_per_w

    def batch(i, carry):
      base = ebase + i * B
      pltpu.sync_copy(dst_h.at[pl.ds(base, B)], dst_v)
      pltpu.sync_copy(cnt_h.at[pl.ds(base, B)], cnt_v)

      def fill(k, kc):
        rows[k, pl.ds(0, L)] = plsc.load_gather(
            cnt_v, [jnp.full((L,), k, jnp.int32)])
        return kc

      lax.fori_loop(0, B, fill, 0)
      pltpu.sync_copy(rows, deg_sh.at[dst_v], add=True)
      return carry

    lax.fori_loop(0, nb, batch, 0)
    plsc.subcore_barrier()

    row0 = c * n + s * rpt
    for z in range(nz):
      pltpu.sync_copy(deg_sh.at[pl.ds(s * rpt + z * ZR, ZR)], rows)
      pltpu.sync_copy(rows, deg_o.at[pl.ds(row0 + z * ZR, ZR)])

  return pl.kernel(
      body,
      out_type=jax.ShapeDtypeStruct((NC * n, d), jnp.float32),
      mesh=mesh,
      scratch_types=[
          pltpu.VMEM((B,), jnp.int32),
          pltpu.VMEM((B,), jnp.float32),
          pltpu.VMEM((B, d), jnp.float32),
          pltpu.VMEM_SHARED((n, d), jnp.float32),
      ],
      compiler_params=pltpu.CompilerParams(needs_layout_passes=False))


def _make_dense(n, d, layer_i, apply_relu, want_emb):
  bn = 512
  nblk = n // bn

  def body(*refs):
    if want_emb:
      (h_ref, p0, p1, d0, d1, w_ref, b_ref, eps_ref, out_ref, emb_ref) = refs
    else:
      (h_ref, p0, p1, d0, d1, w_ref, b_ref, eps_ref, out_ref) = refs
    alpha = jnp.maximum(1.0 + eps_ref[layer_i], 0.0)
    deg = d0[:, 0:1] + d1[:, 0:1]
    denom = jnp.maximum(deg, 1.0)
    t = h_ref[...] * alpha + (p0[...] + p1[...]) / denom
    o = jnp.dot(t, w_ref[...], preferred_element_type=jnp.float32) + b_ref[...]
    if apply_relu:
      o = jnp.maximum(o, 0.0)
    out_ref[...] = o
    if want_emb:
      emb_ref[...] = t

  out_shape = [jax.ShapeDtypeStruct((n, d), jnp.float32)]
  out_specs = [pl.BlockSpec((bn, d), lambda i: (i, 0))]
  if want_emb:
    out_shape.append(jax.ShapeDtypeStruct((n, d), jnp.float32))
    out_specs.append(pl.BlockSpec((bn, d), lambda i: (i, 0)))

  return pl.pallas_call(
      body,
      grid=(nblk,),
      in_specs=[
          pl.BlockSpec((bn, d), lambda i: (i, 0)),
          pl.BlockSpec((bn, d), lambda i: (i, 0)),
          pl.BlockSpec((bn, d), lambda i: (i + nblk, 0)),
          pl.BlockSpec((bn, d), lambda i: (i, 0)),
          pl.BlockSpec((bn, d), lambda i: (i + nblk, 0)),
          pl.BlockSpec((d, d), lambda i: (0, 0)),
          pl.BlockSpec((1, d), lambda i: (0, 0)),
          pl.BlockSpec(memory_space=pltpu.SMEM),
      ],
      out_specs=out_specs,
      out_shape=out_shape,
  )


def kernel(h, edge_index, edge_weights, W0, b0, W1, b1, W2, b2, eps):
  n, d = h.shape
  e = edge_index.shape[1]
  npad = -(-n // 1024) * 1024
  eb = NW * B
  epad = -(-e // eb) * eb
  pad = epad - e

  src = edge_index[0]
  dst = edge_index[1]
  ipad = jnp.zeros((pad,), jnp.int32)
  srcp = jnp.concatenate([src, ipad])
  dstp = jnp.concatenate([dst, ipad])
  ewp = jnp.concatenate([edge_weights.astype(jnp.float32),
                         jnp.zeros((pad,), jnp.float32)])
  cntp = jnp.concatenate([jnp.ones((e,), jnp.float32),
                          jnp.zeros((pad,), jnp.float32)])

  hp = jnp.concatenate([h, jnp.zeros((npad - n, d), jnp.float32)])

  msg = _make_msgpass(npad, d, epad)
  deg2 = _make_degree(npad, d, epad)(dstp, cntp)

  pooled = msg(hp, srcp, dstp, ewp)
  h1 = _make_dense(npad, d, 0, True, False)(
      hp, pooled, pooled, deg2, deg2, W0, b0.reshape(1, d), eps)[0]
  pooled = msg(h1, srcp, dstp, ewp)
  h2 = _make_dense(npad, d, 1, True, False)(
      h1, pooled, pooled, deg2, deg2, W1, b1.reshape(1, d), eps)[0]
  pooled = msg(h2, srcp, dstp, ewp)
  h3, emb = _make_dense(npad, d, 2, False, True)(
      h2, pooled, pooled, deg2, deg2, W2, b2.reshape(1, d), eps)
  return (emb[:n], h3[:n])
